# baseline (speedup 1.0000x reference)
import jax
import jax.numpy as jnp
from jax import lax
from jax.experimental import pallas as pl
from jax.experimental.pallas import tpu as pltpu
from jax.experimental.pallas import tpu_sc as plsc

ENT = 1_000_000
BATCH = 16384
DIM = 64
_INFO = plsc.get_sparse_core_info()
NC, NS, NL = _INFO.num_cores, _INFO.num_subcores, _INFO.num_lanes
NW = NC * NS
B_PER_W = BATCH // NW
IDX_CHUNK = 128
N_CHUNKS = B_PER_W // IDX_CHUNK

COLSUM_BLOCK = 65536


def _colsum_body(x_ref, o_ref):
    o_ref[...] = -jnp.sum(x_ref[...], axis=0)


def _gather_body(colsum_hbm, idx_hbm, out_hbm, idx_v, vals_v, sem):
    wid = lax.axis_index("s") * NC + lax.axis_index("c")
    base = wid * B_PER_W

    for j in range(N_CHUNKS):
        pltpu.sync_copy(idx_hbm.at[pl.ds(base + j * IDX_CHUNK, IDX_CHUNK)],
                        idx_v.at[j])

    copies = []
    for j in range(N_CHUNKS):
        copies.append(pltpu.async_copy(
            colsum_hbm.at[idx_v.at[j]],
            vals_v.at[pl.ds(j * IDX_CHUNK, IDX_CHUNK)],
            sem))
    for c in copies:
        c.wait()

    pltpu.sync_copy(vals_v, out_hbm.at[pl.ds(base, B_PER_W)])


@jax.jit
def _score(ent_embeddings, batch_h):
    table_t = ent_embeddings.T

    colsum = pl.pallas_call(
        _colsum_body,
        out_shape=jax.ShapeDtypeStruct((ENT,), jnp.float32),
        grid=(pl.cdiv(ENT, COLSUM_BLOCK),),
        in_specs=[pl.BlockSpec((DIM, COLSUM_BLOCK), lambda i: (0, i))],
        out_specs=pl.BlockSpec((COLSUM_BLOCK,), lambda i: (i,)),
    )(table_t)

    mesh = plsc.VectorSubcoreMesh(core_axis_name="c", subcore_axis_name="s")
    run = pl.kernel(
        _gather_body,
        out_type=jax.ShapeDtypeStruct((BATCH,), jnp.float32),
        mesh=mesh,
        scratch_types=[
            pltpu.VMEM((N_CHUNKS, IDX_CHUNK), jnp.int32),
            pltpu.VMEM((B_PER_W,), jnp.float32),
            pltpu.SemaphoreType.DMA,
        ],
    )
    return run(colsum, batch_h)


def kernel(ent_embeddings, rel_matrices, batch_h, batch_t, batch_r):
    return _score(ent_embeddings, batch_h)
